# Initial kernel scaffold; baseline (speedup 1.0000x reference)
import jax
import jax.numpy as jnp
from jax.experimental import pallas as pl


def kernel(x, edge_index, bn_gamma, bn_beta, W1, as1, ad1, b1, p1, W2, as2, ad2, b2, p2, W3, as3, ad3, b3):
    raise NotImplementedError("write your pallas kernel here")


# speedup vs baseline: 11.2647x; 11.2647x over previous
import functools
import jax
import jax.numpy as jnp
import numpy as np
from jax import lax
from jax.experimental import pallas as pl
from jax.experimental.pallas import tpu as pltpu
from jax.experimental.pallas import tpu_sc as plsc

N = 10000
E_RAW = 320000
NP = 10240
NW = 32
CH = 128
K_CH = 81
EP = NW * K_CH * CH
HP = 128

f32 = jnp.float32
i32 = jnp.int32

_mesh = plsc.VectorSubcoreMesh(core_axis_name="c", subcore_axis_name="s")


def _sc_gather(table, idx, chunk):
    T, D = table.shape
    per_w = EP // NW
    nch = per_w // chunk

    @functools.partial(
        pl.kernel, mesh=_mesh,
        out_type=jax.ShapeDtypeStruct((EP, D), f32),
        scratch_types=[
            pltpu.VMEM((chunk,), i32),
            pltpu.VMEM((chunk, D), f32),
            pltpu.SemaphoreType.DMA,
        ],
    )
    def k(table_hbm, idx_hbm, out_hbm, idx_v, rows_v, sem):
        wid = lax.axis_index("s") * 2 + lax.axis_index("c")

        def body(j, carry):
            base = wid * per_w + j * chunk
            pltpu.sync_copy(idx_hbm.at[pl.ds(base, chunk)], idx_v)
            pltpu.async_copy(table_hbm.at[idx_v], rows_v, sem).wait()
            pltpu.sync_copy(rows_v, out_hbm.at[pl.ds(base, chunk)])
            return carry

        lax.fori_loop(0, nch, body, 0)

    return k(table, idx)


def _sc_scatter_add(vals, idx3, zeros):
    D = vals.shape[1]
    per_w = K_CH * CH
    rp = NP // 16

    @functools.partial(
        pl.kernel, mesh=_mesh,
        out_type=jax.ShapeDtypeStruct((2, NP, D), f32),
        scratch_types=[
            pltpu.VMEM((CH, D), f32),
            pltpu.VMEM((K_CH, CH), i32),
            pltpu.VMEM_SHARED((NP, D), f32),
            pltpu.SemaphoreType.DMA,
        ],
    )
    def k(vals_hbm, idx_hbm, zeros_hbm, out_hbm, vals_v, idx_v, shared, sem):
        cid = lax.axis_index("c")
        sid = lax.axis_index("s")
        w = cid * 16 + sid
        pltpu.sync_copy(zeros_hbm.at[pl.ds(sid * rp, rp)],
                        shared.at[pl.ds(sid * rp, rp)])
        plsc.subcore_barrier()
        pltpu.sync_copy(idx_hbm.at[w], idx_v)

        def body(j, carry):
            base = w * per_w + j * CH
            pltpu.sync_copy(vals_hbm.at[pl.ds(base, CH)], vals_v)
            pltpu.sync_copy(vals_v, shared.at[idx_v.at[j]], add=True)
            return carry

        lax.fori_loop(0, K_CH, body, 0)
        plsc.subcore_barrier()
        pltpu.sync_copy(shared.at[pl.ds(sid * rp, rp)],
                        out_hbm.at[cid].at[pl.ds(sid * rp, rp)])

    return k(vals, idx3, zeros)


def _bn_body(x_ref, g_ref, b_ref, o_ref):
    x = x_ref[...]
    mu = jnp.mean(x, axis=0, keepdims=True)
    var = jnp.mean((x - mu) * (x - mu), axis=0, keepdims=True)
    o_ref[...] = (x - mu) * lax.rsqrt(var + 1e-5) * g_ref[...] + b_ref[...]


def _tc_bn(x, gamma, beta):
    return pl.pallas_call(
        _bn_body, out_shape=jax.ShapeDtypeStruct(x.shape, f32),
    )(x, gamma.reshape(1, -1), beta.reshape(1, -1))


def _mm_al_body(x_ref, wt_ref, as_ref, ad_ref, xp_ref, als_ref, ald_ref):
    xp = jnp.dot(x_ref[...], wt_ref[...], preferred_element_type=f32)
    xp_ref[...] = xp
    als_ref[...] = jnp.dot(xp, as_ref[...], preferred_element_type=f32)
    ald_ref[...] = jnp.dot(xp, ad_ref[...], preferred_element_type=f32)


def _tc_mm_al(x, Wt, As, Ad, bm):
    n, d_in = x.shape
    F = Wt.shape[1]
    grid = n // bm
    return pl.pallas_call(
        _mm_al_body,
        grid=(grid,),
        in_specs=[
            pl.BlockSpec((bm, d_in), lambda i: (i, 0)),
            pl.BlockSpec((d_in, F), lambda i: (0, 0)),
            pl.BlockSpec((F, HP), lambda i: (0, 0)),
            pl.BlockSpec((F, HP), lambda i: (0, 0)),
        ],
        out_specs=[
            pl.BlockSpec((bm, F), lambda i: (i, 0)),
            pl.BlockSpec((bm, HP), lambda i: (i, 0)),
            pl.BlockSpec((bm, HP), lambda i: (i, 0)),
        ],
        out_shape=[
            jax.ShapeDtypeStruct((n, F), f32),
            jax.ShapeDtypeStruct((n, HP), f32),
            jax.ShapeDtypeStruct((n, HP), f32),
        ],
    )(x, Wt, As, Ad)


def _ex_body(gs_ref, gd_ref, o_ref):
    a = gs_ref[...] + gd_ref[...]
    a = jnp.where(a >= 0, a, 0.2 * a)
    o_ref[...] = jnp.exp(a)


def _tc_ex(gs, gd):
    bm = 4096
    return pl.pallas_call(
        _ex_body,
        grid=(EP // bm,),
        in_specs=[pl.BlockSpec((bm, HP), lambda i: (i, 0))] * 2,
        out_specs=pl.BlockSpec((bm, HP), lambda i: (i, 0)),
        out_shape=jax.ShapeDtypeStruct((EP, HP), f32),
    )(gs, gd)


def _addp_body(p_ref, o_ref):
    o_ref[...] = p_ref[0] + p_ref[1]


def _tc_add_partials(p):
    _, n, D = p.shape
    return pl.pallas_call(
        _addp_body, out_shape=jax.ShapeDtypeStruct((n, D), f32),
    )(p)


def _w_body(ex_ref, gden_ref, o_ref):
    o_ref[...] = ex_ref[...] / (gden_ref[...] + 1e-16)


def _tc_w(ex, gden):
    bm = 4096
    return pl.pallas_call(
        _w_body,
        grid=(EP // bm,),
        in_specs=[pl.BlockSpec((bm, HP), lambda i: (i, 0))] * 2,
        out_specs=pl.BlockSpec((bm, HP), lambda i: (i, 0)),
        out_shape=jax.ShapeDtypeStruct((EP, HP), f32),
    )(ex, gden)


def _v_body(gxp_ref, w_ref, r_ref, o_ref):
    wf = jnp.dot(w_ref[...], r_ref[...], preferred_element_type=f32)
    o_ref[...] = gxp_ref[...] * wf


def _tc_v(gxp, w, R):
    F = gxp.shape[1]
    bm = 2048
    return pl.pallas_call(
        _v_body,
        grid=(EP // bm,),
        in_specs=[
            pl.BlockSpec((bm, F), lambda i: (i, 0)),
            pl.BlockSpec((bm, HP), lambda i: (i, 0)),
            pl.BlockSpec((HP, F), lambda i: (0, 0)),
        ],
        out_specs=pl.BlockSpec((bm, F), lambda i: (i, 0)),
        out_shape=jax.ShapeDtypeStruct((EP, F), f32),
    )(gxp, w, R)


def _post_body(p_ref, b_ref, pr_ref, o_ref):
    h = p_ref[0] + p_ref[1] + b_ref[...]
    o_ref[...] = jnp.where(h >= 0, h, pr_ref[0, 0] * h)


def _tc_post(p, b, prelu, bm):
    F = p.shape[2]
    return pl.pallas_call(
        _post_body,
        grid=(N // bm,),
        in_specs=[
            pl.BlockSpec((2, bm, F), lambda i: (0, i, 0)),
            pl.BlockSpec((1, F), lambda i: (0, 0)),
            pl.BlockSpec((1, 1), lambda i: (0, 0)),
        ],
        out_specs=pl.BlockSpec((bm, F), lambda i: (i, 0)),
        out_shape=jax.ShapeDtypeStruct((N, F), f32),
    )(p, b.reshape(1, -1), prelu.reshape(1, 1))


def _final_body(p_ref, r_ref, b_ref, o_ref):
    z = jnp.dot(p_ref[0] + p_ref[1], r_ref[...],
                preferred_element_type=f32) + b_ref[...]
    m = jnp.max(z, axis=1, keepdims=True)
    s = jnp.log(jnp.sum(jnp.exp(z - m), axis=1, keepdims=True))
    o_ref[...] = z - m - s


def _tc_final(p, R3, b3, bm):
    F = p.shape[2]
    D = R3.shape[1]
    return pl.pallas_call(
        _final_body,
        grid=(N // bm,),
        in_specs=[
            pl.BlockSpec((2, bm, F), lambda i: (0, i, 0)),
            pl.BlockSpec((F, D), lambda i: (0, 0)),
            pl.BlockSpec((1, D), lambda i: (0, 0)),
        ],
        out_specs=pl.BlockSpec((bm, D), lambda i: (i, 0)),
        out_shape=jax.ShapeDtypeStruct((N, D), f32),
    )(p, R3, b3.reshape(1, -1))


def _blockdiag(a):
    H, C = a.shape
    r = np.arange(H * C)
    return jnp.zeros((H * C, HP), f32).at[r, r // C].set(a.reshape(-1))


def _repeat_mat(H, C):
    m = np.zeros((HP, H * C), dtype=np.float32)
    for h in range(H):
        m[h, h * C:(h + 1) * C] = 1.0
    return jnp.asarray(m)


def _pad_table(a, cols):
    return jnp.pad(a, ((0, NP - a.shape[0]), (0, cols - a.shape[1])))


def _gat_layer(x_feat, src, dst, dst3, Wt, As, Ad, zeros_hp, H, C, bm):
    F = H * C
    Fp = ((F + 127) // 128) * 128
    xp, als, ald = _tc_mm_al(x_feat, Wt, As, Ad, bm)
    gs = _sc_gather(_pad_table(als, HP), src, CH)
    gd = _sc_gather(_pad_table(ald, HP), dst, CH)
    ex = _tc_ex(gs, gd)
    den = _tc_add_partials(_sc_scatter_add(ex, dst3, zeros_hp))
    gden = _sc_gather(den, dst, CH)
    w = _tc_w(ex, gden)
    chunk = CH if Fp <= 512 else 64
    gxp = _sc_gather(_pad_table(xp, Fp), src, chunk)
    R = jnp.pad(_repeat_mat(H, C), ((0, 0), (0, Fp - F)))
    V = _tc_v(gxp, w, R)
    zp = jnp.zeros((NP, 128), f32)
    panels = [_sc_scatter_add(V[:, p0:p0 + 128], dst3, zp)
              for p0 in range(0, Fp, 128)]
    res = jnp.concatenate(panels, axis=2) if len(panels) > 1 else panels[0]
    return res[:, :, :F]


def kernel(x, edge_index, bn_gamma, bn_beta, W1, as1, ad1, b1, p1,
           W2, as2, ad2, b2, p2, W3, as3, ad3, b3):
    ar = jnp.arange(N, dtype=i32)
    padi = jnp.full((EP - E_RAW - N,), N, dtype=i32)
    src = jnp.concatenate([edge_index[0].astype(i32), ar, padi])
    dst = jnp.concatenate([edge_index[1].astype(i32), ar, padi])
    dst3 = dst.reshape(NW, K_CH, CH)
    zeros_hp = jnp.zeros((NP, HP), f32)

    x_in = _tc_bn(x, bn_gamma, bn_beta)

    h1p = _gat_layer(x_in, src, dst, dst3, W1.T, _blockdiag(as1),
                     _blockdiag(ad1), zeros_hp, 16, 64, 1000)
    h1 = _tc_post(h1p, b1, p1, 1000)

    x2 = jnp.concatenate([x_in, h1], axis=1)
    h2p = _gat_layer(x2, src, dst, dst3, W2.T, _blockdiag(as2),
                     _blockdiag(ad2), zeros_hp, 8, 8, 1000)
    h2 = _tc_post(h2p, b2, p2, 1000)

    h3p = _gat_layer(h2, src, dst, dst3, W3.T, _blockdiag(as3),
                     _blockdiag(ad3), zeros_hp, 8, 64, 1000)
    m3 = np.zeros((512, 64), dtype=np.float32)
    for h in range(8):
        m3[h * 64:(h + 1) * 64, :] = np.eye(64, dtype=np.float32) / 8.0
    return _tc_final(h3p, jnp.asarray(m3), b3, 1000)
